# baseline (speedup 1.0000x reference)
import jax
import jax.numpy as jnp
import numpy as np
from jax.experimental import pallas as pl
from jax.experimental.pallas import tpu as pltpu

B = 16; T = 168; N = 512; D = 128; H = 4; DK = 32; M = 384; K = 8
_SC = 1.0 / np.sqrt(DK)
_FLAT = N * D
_NC = 64
_CHUNK = _NC * D
_NCHUNK = _FLAT // _CHUNK


def _ln(x, g, b):
    mu = jnp.mean(x, axis=-1, keepdims=True)
    v = jnp.mean((x - mu) ** 2, axis=-1, keepdims=True)
    return (x - mu) * jax.lax.rsqrt(v + 1e-5) * g + b


def _gelu(x):
    return 0.5 * x * (1.0 + jax.lax.erf(x * np.float32(1.0 / np.sqrt(2.0))))


def _head_mask():
    i = jax.lax.broadcasted_iota(jnp.int32, (D, D), 0) // DK
    j = jax.lax.broadcasted_iota(jnp.int32, (D, D), 1) // DK
    return jnp.where(i == j, 1.0, 0.0).astype(jnp.float32)


def _seg_softmax(x, mask):
    m = jnp.max(x, axis=-1, keepdims=True)
    e = jnp.exp(x - m)
    s = jnp.dot(e, mask, preferred_element_type=jnp.float32)
    return e / s


def _enc_body(x_ref, fW_ref, fb_ref, cmat_ref, eg_ref, eb_ref, pb_ref,
              n1g_ref, n1b_ref, n2g_ref, n2b_ref,
              qW_ref, qb_ref, kW_ref, kb_ref, vW_ref, vb_ref, oW_ref, ob_ref,
              w1_ref, b1_ref, w2_ref, b2_ref, out_ref):
    mask = _head_mask()
    xb = x_ref[0]
    h0 = jax.lax.dot_general(xb, fW_ref[...], (((0,), (0,)), ((), ())),
                             preferred_element_type=jnp.float32) + fb_ref[...]
    hs = jnp.dot(h0, cmat_ref[...], preferred_element_type=jnp.float32)
    h = _ln(_gelu(hs), eg_ref[...], eb_ref[...])
    pb = pb_ref[...]
    scale_p = pb[:, D:2 * D]
    mem_key = jax.nn.sigmoid(pb[:, :D]) * pb[:, 2 * D:]
    h1 = _ln(h, n1g_ref[...], n1b_ref[...]) * (1.0 + scale_p)
    q = jnp.dot(h1, qW_ref[...], preferred_element_type=jnp.float32) + qb_ref[...]
    k = jnp.dot(h1, kW_ref[...], preferred_element_type=jnp.float32) + kb_ref[...]
    v = jnp.dot(h1, vW_ref[...], preferred_element_type=jnp.float32) + vb_ref[...]
    qs = _seg_softmax(q * _SC, mask)
    ksm = _seg_softmax(k * _SC, mask) + _seg_softmax(mem_key * _SC, mask)
    kv = jax.lax.dot_general(ksm, v, (((0,), (0,)), ((), ())),
                             preferred_element_type=jnp.float32)
    attn = jnp.dot(qs, kv * mask, preferred_element_type=jnp.float32)
    attn = jnp.dot(attn, oW_ref[...], preferred_element_type=jnp.float32) + ob_ref[...]
    h2 = h + attn
    t = _ln(h2, n2g_ref[...], n2b_ref[...])
    f = _gelu(jnp.dot(t, w1_ref[...], preferred_element_type=jnp.float32) + b1_ref[...])
    out_ref[0] = h2 + jnp.dot(f, w2_ref[...], preferred_element_type=jnp.float32) + b2_ref[...]


def _sim_body(hf_ref, mf_ref, topi_ref, dots_ref, msq_ref):
    c = pl.program_id(0)

    @pl.when(c == 0)
    def _():
        dots_ref[...] = jnp.zeros_like(dots_ref)
        msq_ref[...] = jnp.zeros_like(msq_ref)

    mf = mf_ref[...]
    dots_acc = dots_ref[...]
    for j in range(_NC):
        hj = hf_ref[:, j, :]
        dots_acc += jax.lax.dot_general(
            hj, mf[:, j * D:(j + 1) * D], (((1,), (1,)), ((), ())),
            preferred_element_type=jnp.float32)
    dots_ref[...] = dots_acc
    msq_ref[...] += jnp.sum(mf * mf, axis=1, keepdims=True)

    @pl.when(c == _NCHUNK - 1)
    def _():
        mn = jnp.sqrt(msq_ref[...]) + 1e-8
        sim = dots_ref[...] / jnp.reshape(mn, (1, M))
        iot = jax.lax.broadcasted_iota(jnp.int32, (B, M), 1)
        for j in range(K):
            mx = jnp.max(sim, axis=1, keepdims=True)
            sel = jnp.where(sim >= mx, iot, jnp.int32(M))
            idx = jnp.min(sel, axis=1, keepdims=True)
            topi_ref[:, j:j + 1] = idx
            sim = jnp.where(iot == idx, -jnp.inf, sim)


def _cross_body(topi_ref, h_ref, s0_ref, s1_ref, s2_ref, s3_ref, s4_ref,
                s5_ref, s6_ref, s7_ref, wqT_ref, bq_ref, wkT_ref, bk_ref,
                wvT_ref, bv_ref, wo_ref, bo_ref, gWh_ref, gWe_ref, gb_ref,
                poW_ref, pob_ref, out_ref):
    mask = _head_mask()
    hb = h_ref[0]
    qp = jnp.dot(hb, wqT_ref[...], preferred_element_type=jnp.float32) + bq_ref[...]
    slots = (s0_ref, s1_ref, s2_ref, s3_ref, s4_ref, s5_ref, s6_ref, s7_ref)
    scores, vals = [], []
    for s_ref in slots:
        slot = s_ref[0]
        kp = jnp.dot(slot, wkT_ref[...], preferred_element_type=jnp.float32) + bk_ref[...]
        vp = jnp.dot(slot, wvT_ref[...], preferred_element_type=jnp.float32) + bv_ref[...]
        scores.append(jnp.dot(qp * kp, mask, preferred_element_type=jnp.float32) * _SC)
        vals.append(vp)
    m = scores[0]
    for s in scores[1:]:
        m = jnp.maximum(m, s)
    den = jnp.zeros_like(m)
    acc = jnp.zeros_like(m)
    for s, vp in zip(scores, vals):
        e = jnp.exp(s - m)
        den += e
        acc += e * vp
    o = acc / den
    o2 = jnp.dot(o, wo_ref[...], preferred_element_type=jnp.float32) + bo_ref[...]
    g = jax.nn.sigmoid(jnp.dot(hb, gWh_ref[...], preferred_element_type=jnp.float32)
                       + jnp.dot(o2, gWe_ref[...], preferred_element_type=jnp.float32)
                       + gb_ref[...])
    fused = g * hb + (1.0 - g) * o2
    out_ref[0] = jnp.dot(fused, poW_ref[...], preferred_element_type=jnp.float32) + pob_ref[...]


def kernel(x, season_labels, year_labels, fconv_W, fconv_b, complex_weight,
           enc_ln_g, enc_ln_b, pattern_bank, n1_g, n1_b, n2_g, n2_b,
           q_W, q_b, k_W, k_b, v_W, v_b, o_W, o_b,
           ffn_W1, ffn_b1, ffn_W2, ffn_b2,
           mha_in_W, mha_in_b, mha_out_W, mha_out_b,
           gate_W, gate_b, outp_W, outp_b, memory_bank):
    f32 = jnp.float32
    row = lambda a: a.reshape(1, -1).astype(f32)

    wc = complex_weight[0, 0, :, 0] + 1j * complex_weight[0, 0, :, 1]
    eyeF = jnp.fft.rfft(jnp.eye(D, dtype=f32), axis=1, norm='ortho')
    cmat = jnp.fft.irfft(eyeF * wc[None, :], n=D, axis=1, norm='ortho').astype(f32)

    mfflat = memory_bank.reshape(M, _FLAT)

    wfull = lambda shape: pl.BlockSpec(shape, lambda b: (0, 0))
    hfin = pl.pallas_call(
        _enc_body,
        grid=(B,),
        in_specs=[
            pl.BlockSpec((1, T, N), lambda b: (b, 0, 0)),
            wfull((T, D)), wfull((1, D)), wfull((D, D)),
            wfull((1, D)), wfull((1, D)),
            wfull((N, 3 * D)),
            wfull((1, D)), wfull((1, D)), wfull((1, D)), wfull((1, D)),
            wfull((D, D)), wfull((1, D)), wfull((D, D)), wfull((1, D)),
            wfull((D, D)), wfull((1, D)), wfull((D, D)), wfull((1, D)),
            wfull((D, 4 * D)), wfull((1, 4 * D)), wfull((4 * D, D)), wfull((1, D)),
        ],
        out_specs=pl.BlockSpec((1, N, D), lambda b: (b, 0, 0)),
        out_shape=jax.ShapeDtypeStruct((B, N, D), f32),
        compiler_params=pltpu.CompilerParams(dimension_semantics=("arbitrary",)),
    )(x, fconv_W, row(fconv_b), cmat, row(enc_ln_g), row(enc_ln_b), pattern_bank,
      row(n1_g), row(n1_b), row(n2_g), row(n2_b),
      q_W, row(q_b), k_W, row(k_b), v_W, row(v_b), o_W, row(o_b),
      ffn_W1, row(ffn_b1), ffn_W2, row(ffn_b2))

    topi = pl.pallas_call(
        _sim_body,
        grid=(_NCHUNK,),
        in_specs=[
            pl.BlockSpec((B, _NC, D), lambda c: (0, c, 0)),
            pl.BlockSpec((M, _CHUNK), lambda c: (0, c)),
        ],
        out_specs=pl.BlockSpec((B, K), lambda c: (0, 0)),
        out_shape=jax.ShapeDtypeStruct((B, K), jnp.int32),
        scratch_shapes=[
            pltpu.VMEM((B, M), f32),
            pltpu.VMEM((M, 1), f32),
        ],
        compiler_params=pltpu.CompilerParams(dimension_semantics=("arbitrary",)),
    )(hfin, mfflat)

    slot_spec = [
        pl.BlockSpec((1, N, D), (lambda j: (lambda b, ti: (ti[b, j], 0, 0)))(j))
        for j in range(K)
    ]
    grid_spec = pltpu.PrefetchScalarGridSpec(
        num_scalar_prefetch=1,
        grid=(B,),
        in_specs=[pl.BlockSpec((1, N, D), lambda b, ti: (b, 0, 0))]
        + slot_spec
        + [pl.BlockSpec((D, D), lambda b, ti: (0, 0)),
           pl.BlockSpec((1, D), lambda b, ti: (0, 0)),
           pl.BlockSpec((D, D), lambda b, ti: (0, 0)),
           pl.BlockSpec((1, D), lambda b, ti: (0, 0)),
           pl.BlockSpec((D, D), lambda b, ti: (0, 0)),
           pl.BlockSpec((1, D), lambda b, ti: (0, 0)),
           pl.BlockSpec((D, D), lambda b, ti: (0, 0)),
           pl.BlockSpec((1, D), lambda b, ti: (0, 0)),
           pl.BlockSpec((D, D), lambda b, ti: (0, 0)),
           pl.BlockSpec((D, D), lambda b, ti: (0, 0)),
           pl.BlockSpec((1, D), lambda b, ti: (0, 0)),
           pl.BlockSpec((D, D), lambda b, ti: (0, 0)),
           pl.BlockSpec((1, D), lambda b, ti: (0, 0))],
        out_specs=pl.BlockSpec((1, N, D), lambda b, ti: (b, 0, 0)),
    )
    out = pl.pallas_call(
        _cross_body,
        grid_spec=grid_spec,
        out_shape=jax.ShapeDtypeStruct((B, N, D), f32),
        compiler_params=pltpu.CompilerParams(dimension_semantics=("arbitrary",)),
    )(topi, hfin, *([memory_bank] * K),
      mha_in_W[:D].T, row(mha_in_b[:D]),
      mha_in_W[D:2 * D].T, row(mha_in_b[D:2 * D]),
      mha_in_W[2 * D:].T, row(mha_in_b[2 * D:]),
      mha_out_W, row(mha_out_b),
      gate_W[:D], gate_W[D:], row(gate_b),
      outp_W, row(outp_b))
    return out
